# baseline (speedup 1.0000x reference)
import functools

import jax
import jax.numpy as jnp
from jax import lax
from jax.experimental import pallas as pl
from jax.experimental.pallas import tpu as pltpu
from jax.experimental.pallas import tpu_sc as plsc

VOCAB = 100000
EMBED_DIM = 32
HIDDEN = 128
BATCH = 1024
CTX = 200

_L = 16
_NW = 32
_DEPTH = 2
_NROT = _DEPTH + 1


def _make_sc_kernel(nch, bpw):
    def _sc_pool_kernel(emb_hbm, idx_hbm, out_hbm, idx_v, pooled_v, *rest):
        nc = 2
        wid = lax.axis_index("s") * nc + lax.axis_index("c")
        base = wid * bpw

        inv = jnp.full((_L,), 1.0 / CTX, dtype=jnp.float32)
        rows = rest[:_NROT]
        isem = rest[_NROT]
        gsems = rest[_NROT + 1 :]

        pltpu.async_copy(idx_hbm.at[wid], idx_v, isem).wait()

        cps = [None] * _NROT

        def start_gather(c):
            cps[c % _NROT] = pltpu.async_copy(
                emb_hbm.at[idx_v.at[c]], rows[c % _NROT], gsems[c % _NROT]
            )

        for c in range(_DEPTH):
            start_gather(c)

        def acc8(buf, lo, hi, a0, a1):
            def body(j, carry):
                b0, b1 = carry
                for u in range(8):
                    r = j * 8 + u
                    b0 = b0 + buf[r, 0:_L]
                    b1 = b1 + buf[r, _L : 2 * _L]
                return (b0, b1)

            return lax.fori_loop(lo // 8, hi // 8, body, (a0, a1))

        z = jnp.zeros((_L,), dtype=jnp.float32)
        a0, a1 = z, z
        for c in range(nch):
            if c + _DEPTH < nch:
                start_gather(c + _DEPTH)
            cps[c % _NROT].wait()
            buf = rows[c % _NROT]
            start = 128 * c
            r = start // CTX
            split = min(128, CTX * (r + 1) - start)
            a0, a1 = acc8(buf, 0, split, a0, a1)
            if split < 128:
                pooled_v[r, 0:_L] = a0 * inv
                pooled_v[r, _L : 2 * _L] = a1 * inv
                a0, a1 = acc8(buf, split, 128, z, z)
            elif (start + 128) % CTX == 0:
                pooled_v[r, 0:_L] = a0 * inv
                pooled_v[r, _L : 2 * _L] = a1 * inv
                a0, a1 = z, z

        pltpu.sync_copy(pooled_v, out_hbm.at[pl.ds(base, bpw)])

    return _sc_pool_kernel


def _sc_pool(emb, idx_chunks, nch, bpw):
    mesh = plsc.VectorSubcoreMesh(core_axis_name="c", subcore_axis_name="s")
    return pl.kernel(
        _make_sc_kernel(nch, bpw),
        mesh=mesh,
        out_type=jax.ShapeDtypeStruct((_NW * bpw, EMBED_DIM), jnp.float32),
        scratch_types=(
            [
                pltpu.VMEM((nch, 128), jnp.int32),
                pltpu.VMEM((bpw, EMBED_DIM), jnp.float32),
            ]
            + [pltpu.VMEM((128, EMBED_DIM), jnp.float32)] * _NROT
            + [pltpu.SemaphoreType.DMA] * (_NROT + 1)
        ),
        compiler_params=pltpu.CompilerParams(use_tc_tiling_on_sc=False),
    )(emb, idx_chunks)


_TW = 2048
_NT = pl.cdiv(VOCAB, _TW)
_TAIL = VOCAB - (_NT - 1) * _TW
_NBUF = 5


def _make_mlp_block(rows, row_base, has_prev):
    def _mlp_block(*args):
        if has_prev:
            (pooled_ref, w1_ref, b1_ref, w2_ref, b2_ref, _prev, out_hbm,
             h_ref, obuf, tailbuf, sems) = args
        else:
            (pooled_ref, w1_ref, b1_ref, w2_ref, b2_ref, out_hbm,
             h_ref, obuf, tailbuf, sems) = args
        j = pl.program_id(0)

        @pl.when(j == 0)
        def _():
            h = (
                jnp.dot(
                    pooled_ref[...], w1_ref[...], preferred_element_type=jnp.float32
                )
                + b1_ref[...]
            )
            h_ref[...] = jnp.maximum(h, 0.0)

        slot = lax.rem(j, _NBUF)

        @pl.when(j >= _NBUF)
        def _():
            pltpu.make_async_copy(
                obuf.at[slot],
                out_hbm.at[pl.ds(row_base, rows), pl.ds((j - _NBUF) * _TW, _TW)],
                sems.at[slot],
            ).wait()

        blk = (
            jnp.dot(h_ref[...], w2_ref[...], preferred_element_type=jnp.float32)
            + b2_ref[...]
        )

        @pl.when(j < _NT - 1)
        def _():
            obuf[slot] = blk
            pltpu.make_async_copy(
                obuf.at[slot],
                out_hbm.at[pl.ds(row_base, rows), pl.ds(j * _TW, _TW)],
                sems.at[slot],
            ).start()

        @pl.when(j == _NT - 1)
        def _():
            tailbuf[...] = blk[:, :_TAIL]
            pltpu.make_async_copy(
                tailbuf,
                out_hbm.at[pl.ds(row_base, rows), pl.ds((_NT - 1) * _TW, _TAIL)],
                sems.at[slot],
            ).start()
            for step in range(_NT - _NBUF, _NT):
                s = step % _NBUF
                if step == _NT - 1:
                    pltpu.make_async_copy(
                        tailbuf,
                        out_hbm.at[pl.ds(row_base, rows), pl.ds(step * _TW, _TAIL)],
                        sems.at[s],
                    ).wait()
                else:
                    pltpu.make_async_copy(
                        obuf.at[s],
                        out_hbm.at[pl.ds(row_base, rows), pl.ds(step * _TW, _TW)],
                        sems.at[s],
                    ).wait()

    return _mlp_block


def _tc_mlp_half(pooled, W1, b1, W2, b2, out_prev, row_base):
    rows = pooled.shape[0]
    has_prev = out_prev is not None
    in_specs = [
        pl.BlockSpec((rows, EMBED_DIM), lambda j: (0, 0)),
        pl.BlockSpec((EMBED_DIM, HIDDEN), lambda j: (0, 0)),
        pl.BlockSpec((1, HIDDEN), lambda j: (0, 0)),
        pl.BlockSpec((HIDDEN, _TW), lambda j: (0, j)),
        pl.BlockSpec((1, _TW), lambda j: (0, j)),
    ]
    operands = [pooled, W1, b1.reshape(1, HIDDEN), W2, b2.reshape(1, VOCAB)]
    kwargs = {}
    if has_prev:
        in_specs.append(pl.BlockSpec(memory_space=pltpu.MemorySpace.HBM))
        operands.append(out_prev)
        kwargs["input_output_aliases"] = {5: 0}
    return pl.pallas_call(
        _make_mlp_block(rows, row_base, has_prev),
        grid=(_NT,),
        in_specs=in_specs,
        out_specs=pl.BlockSpec(memory_space=pltpu.MemorySpace.HBM),
        out_shape=jax.ShapeDtypeStruct((BATCH, VOCAB), jnp.float32),
        scratch_shapes=[
            pltpu.VMEM((rows, HIDDEN), jnp.float32),
            pltpu.VMEM((_NBUF, rows, _TW), jnp.float32),
            pltpu.VMEM((rows, _TAIL), jnp.float32),
            pltpu.SemaphoreType.DMA((_NBUF,)),
        ],
        compiler_params=pltpu.CompilerParams(
            dimension_semantics=("arbitrary",),
        ),
        **kwargs,
    )(*operands)


def kernel(inputs, emb, W1, b1, W2, b2):
    idx = inputs.astype(jnp.int32)
    half = BATCH // 2
    bpw = half // _NW
    nch = bpw * CTX // 128
    idx_a = idx[:half].reshape(_NW, nch, 128)
    idx_b = idx[half:].reshape(_NW, nch, 128)
    pooled_a = _sc_pool(emb, idx_a, nch, bpw)
    pooled_b = _sc_pool(emb, idx_b, nch, bpw)
    out = _tc_mlp_half(pooled_a, W1, b1, W2, b2, None, 0)
    return _tc_mlp_half(pooled_b, W1, b1, W2, b2, out, half)


# speedup vs baseline: 1.0529x; 1.0529x over previous
import functools

import jax
import jax.numpy as jnp
from jax import lax
from jax.experimental import pallas as pl
from jax.experimental.pallas import tpu as pltpu
from jax.experimental.pallas import tpu_sc as plsc

VOCAB = 100000
EMBED_DIM = 32
HIDDEN = 128
BATCH = 1024
CTX = 200

_L = 16
_NW = 32
_DEPTH = 2
_NROT = _DEPTH + 1


def _make_sc_kernel(nch, bpw):
    def _sc_pool_kernel(emb_hbm, idx_hbm, out_hbm, idx_v, pooled_v, *rest):
        nc = 2
        wid = lax.axis_index("s") * nc + lax.axis_index("c")
        base = wid * bpw

        inv = jnp.full((_L,), 1.0 / CTX, dtype=jnp.float32)
        rows = rest[:_NROT]
        isem = rest[_NROT]
        gsems = rest[_NROT + 1 :]

        pltpu.async_copy(idx_hbm.at[wid], idx_v, isem).wait()

        cps = [None] * _NROT

        def start_gather(c):
            cps[c % _NROT] = pltpu.async_copy(
                emb_hbm.at[idx_v.at[c]], rows[c % _NROT], gsems[c % _NROT]
            )

        for c in range(_DEPTH):
            start_gather(c)

        def acc8(buf, lo, hi, acc):
            def body(j, carry):
                a = list(carry)
                for u in range(8):
                    r = j * 8 + u
                    ch = u % 4
                    a[2 * ch] = a[2 * ch] + buf[r, 0:_L]
                    a[2 * ch + 1] = a[2 * ch + 1] + buf[r, _L : 2 * _L]
                return tuple(a)

            return lax.fori_loop(lo // 8, hi // 8, body, acc)

        z = jnp.zeros((_L,), dtype=jnp.float32)
        zacc = (z,) * 8

        def flush(acc, r):
            pooled_v[r, 0:_L] = ((acc[0] + acc[2]) + (acc[4] + acc[6])) * inv
            pooled_v[r, _L : 2 * _L] = ((acc[1] + acc[3]) + (acc[5] + acc[7])) * inv

        acc = zacc
        for c in range(nch):
            if c + _DEPTH < nch:
                start_gather(c + _DEPTH)
            cps[c % _NROT].wait()
            buf = rows[c % _NROT]
            start = 128 * c
            r = start // CTX
            split = min(128, CTX * (r + 1) - start)
            acc = acc8(buf, 0, split, acc)
            if split < 128:
                flush(acc, r)
                acc = acc8(buf, split, 128, zacc)
            elif (start + 128) % CTX == 0:
                flush(acc, r)
                acc = zacc

        pltpu.sync_copy(pooled_v, out_hbm.at[pl.ds(base, bpw)])

    return _sc_pool_kernel


def _sc_pool(emb, idx_chunks, nch, bpw):
    mesh = plsc.VectorSubcoreMesh(core_axis_name="c", subcore_axis_name="s")
    return pl.kernel(
        _make_sc_kernel(nch, bpw),
        mesh=mesh,
        out_type=jax.ShapeDtypeStruct((_NW * bpw, EMBED_DIM), jnp.float32),
        scratch_types=(
            [
                pltpu.VMEM((nch, 128), jnp.int32),
                pltpu.VMEM((bpw, EMBED_DIM), jnp.float32),
            ]
            + [pltpu.VMEM((128, EMBED_DIM), jnp.float32)] * _NROT
            + [pltpu.SemaphoreType.DMA] * (_NROT + 1)
        ),
        compiler_params=pltpu.CompilerParams(use_tc_tiling_on_sc=False),
    )(emb, idx_chunks)


_TW = 2048
_NT = pl.cdiv(VOCAB, _TW)
_TAIL = VOCAB - (_NT - 1) * _TW
_NBUF = 5


def _make_mlp_block(rows, row_base, has_prev):
    def _mlp_block(*args):
        if has_prev:
            (pooled_ref, w1_ref, b1_ref, w2_ref, b2_ref, _prev, out_hbm,
             h_ref, obuf, tailbuf, sems) = args
        else:
            (pooled_ref, w1_ref, b1_ref, w2_ref, b2_ref, out_hbm,
             h_ref, obuf, tailbuf, sems) = args
        j = pl.program_id(0)

        @pl.when(j == 0)
        def _():
            h = (
                jnp.dot(
                    pooled_ref[...], w1_ref[...], preferred_element_type=jnp.float32
                )
                + b1_ref[...]
            )
            h_ref[...] = jnp.maximum(h, 0.0)

        slot = lax.rem(j, _NBUF)

        @pl.when(j >= _NBUF)
        def _():
            pltpu.make_async_copy(
                obuf.at[slot],
                out_hbm.at[pl.ds(row_base, rows), pl.ds((j - _NBUF) * _TW, _TW)],
                sems.at[slot],
            ).wait()

        blk = (
            jnp.dot(h_ref[...], w2_ref[...], preferred_element_type=jnp.float32)
            + b2_ref[...]
        )

        @pl.when(j < _NT - 1)
        def _():
            obuf[slot] = blk
            pltpu.make_async_copy(
                obuf.at[slot],
                out_hbm.at[pl.ds(row_base, rows), pl.ds(j * _TW, _TW)],
                sems.at[slot],
            ).start()

        @pl.when(j == _NT - 1)
        def _():
            tailbuf[...] = blk[:, :_TAIL]
            pltpu.make_async_copy(
                tailbuf,
                out_hbm.at[pl.ds(row_base, rows), pl.ds((_NT - 1) * _TW, _TAIL)],
                sems.at[slot],
            ).start()
            for step in range(_NT - _NBUF, _NT):
                s = step % _NBUF
                if step == _NT - 1:
                    pltpu.make_async_copy(
                        tailbuf,
                        out_hbm.at[pl.ds(row_base, rows), pl.ds(step * _TW, _TAIL)],
                        sems.at[s],
                    ).wait()
                else:
                    pltpu.make_async_copy(
                        obuf.at[s],
                        out_hbm.at[pl.ds(row_base, rows), pl.ds(step * _TW, _TW)],
                        sems.at[s],
                    ).wait()

    return _mlp_block


def _tc_mlp_half(pooled, W1, b1, W2, b2, out_prev, row_base):
    rows = pooled.shape[0]
    has_prev = out_prev is not None
    in_specs = [
        pl.BlockSpec((rows, EMBED_DIM), lambda j: (0, 0)),
        pl.BlockSpec((EMBED_DIM, HIDDEN), lambda j: (0, 0)),
        pl.BlockSpec((1, HIDDEN), lambda j: (0, 0)),
        pl.BlockSpec((HIDDEN, _TW), lambda j: (0, j)),
        pl.BlockSpec((1, _TW), lambda j: (0, j)),
    ]
    operands = [pooled, W1, b1.reshape(1, HIDDEN), W2, b2.reshape(1, VOCAB)]
    kwargs = {}
    if has_prev:
        in_specs.append(pl.BlockSpec(memory_space=pltpu.MemorySpace.HBM))
        operands.append(out_prev)
        kwargs["input_output_aliases"] = {5: 0}
    return pl.pallas_call(
        _make_mlp_block(rows, row_base, has_prev),
        grid=(_NT,),
        in_specs=in_specs,
        out_specs=pl.BlockSpec(memory_space=pltpu.MemorySpace.HBM),
        out_shape=jax.ShapeDtypeStruct((BATCH, VOCAB), jnp.float32),
        scratch_shapes=[
            pltpu.VMEM((rows, HIDDEN), jnp.float32),
            pltpu.VMEM((_NBUF, rows, _TW), jnp.float32),
            pltpu.VMEM((rows, _TAIL), jnp.float32),
            pltpu.SemaphoreType.DMA((_NBUF,)),
        ],
        compiler_params=pltpu.CompilerParams(
            dimension_semantics=("arbitrary",),
        ),
        **kwargs,
    )(*operands)


def kernel(inputs, emb, W1, b1, W2, b2):
    idx = inputs.astype(jnp.int32)
    bpw = BATCH // _NW
    nch = bpw * CTX // 128
    idx_chunks = idx.reshape(_NW, nch, 128)
    pooled = _sc_pool(emb, idx_chunks, nch, bpw)
    return _tc_mlp_half(pooled, W1, b1, W2, b2, None, 0)
